# baseline (speedup 1.0000x reference)
import jax
import jax.numpy as jnp
from jax.experimental import pallas as pl
from jax.experimental.pallas import tpu as pltpu

B = 16
T = 32768
BAG = 128
CLS = 10
FEAT_ROOT = 192
MASK_ROOT = 2
FEAT_SUB = 128
MASK_SUB = 4

NB = 4
BT = T // NB

_bf = jnp.bfloat16
_f32 = jnp.float32


def _onehot_rows(i, cu_ref):
    t_row = jax.lax.broadcasted_iota(jnp.int32, (1, BT), 1) + i * BT
    io = jax.lax.broadcasted_iota(jnp.int32, (B, 1), 0)
    lo = jnp.zeros((B, 1), jnp.int32)
    hi = jnp.zeros((B, 1), jnp.int32)
    for s in range(B):
        lo = jnp.where(io == s, cu_ref[s], lo)
        hi = jnp.where(io == s, cu_ref[s + 1], hi)
    return (lo <= t_row) & (t_row < hi)


def _pass1_body(sub_f_ref, sub_mT_ref, cu_ref, Wsub_ref, bsub_ref, Wos2_ref,
                sums_ref, partT_ref, acc_ref):
    i = pl.program_id(0)
    x = sub_f_ref[...].astype(_bf)
    mi = jnp.transpose(sub_mT_ref[...])
    xs = (jnp.dot(x, Wsub_ref[0:FEAT_SUB, :].astype(_bf),
                  preferred_element_type=_f32)
          + jnp.dot(mi.astype(_bf), Wsub_ref[FEAT_SUB:, :].astype(_bf),
                    preferred_element_type=_f32)
          + bsub_ref[...])
    x_raw = jnp.maximum(xs, 0.01 * xs)
    x_raw_b = x_raw.astype(_bf)

    p = jnp.dot(x_raw_b, Wos2_ref[...].astype(_bf),
                preferred_element_type=_f32).astype(_bf)
    partT_ref[...] = jnp.transpose(jnp.where(mi >= 1, _bf(-jnp.inf), p))

    onehot = _onehot_rows(i, cu_ref).astype(_bf)
    contrib = jnp.dot(onehot, x_raw_b, preferred_element_type=_f32)

    @pl.when(i == 0)
    def _():
        acc_ref[...] = contrib

    @pl.when(i > 0)
    def _():
        acc_ref[...] = acc_ref[...] + contrib

    @pl.when(i == NB - 1)
    def _():
        sums_ref[...] = acc_ref[...]


def _root_body(sums_ref, cu_ref, root_f_ref, root_m_ref,
               gsub_ref, besub_ref, Wroot_ref, broot_ref, groot_ref, beroot_ref,
               Wv_ref, bv_ref, Wcp_ref, bcp_ref, Wca_ref, bca_ref,
               Wor_ref, bor_ref, Wos1_ref, bos_ref,
               o_v_ref, o_cls_p_ref, root_cat_ref, r4_ref):
    io = jax.lax.broadcasted_iota(jnp.int32, (B, 1), 0)
    denom = jnp.ones((B, 1), _f32)
    nz = jnp.zeros((B, 1), _f32)
    for s in range(B):
        l = cu_ref[s + 1] - cu_ref[s]
        denom = jnp.where(io == s, jnp.maximum(l, 1).astype(_f32), denom)
        nz = jnp.where(io == s, (l > 0).astype(_f32), nz)
    x_agg = sums_ref[...] / denom * nz

    mu = jnp.mean(x_agg, axis=1, keepdims=True)
    var = jnp.mean((x_agg - mu) ** 2, axis=1, keepdims=True)
    x_agg = (x_agg - mu) / jnp.sqrt(var + 1e-5) * gsub_ref[...] + besub_ref[...]

    rm = root_m_ref[...].astype(_f32)
    xr = (jnp.dot(x_agg, Wroot_ref[0:BAG, :], preferred_element_type=_f32)
          + jnp.dot(root_f_ref[:, BAG:], Wroot_ref[BAG:FEAT_ROOT, :],
                    preferred_element_type=_f32)
          + jnp.dot(rm, Wroot_ref[FEAT_ROOT:, :], preferred_element_type=_f32)
          + broot_ref[...])
    xr = jnp.maximum(xr, 0.01 * xr)
    mu2 = jnp.mean(xr, axis=1, keepdims=True)
    var2 = jnp.mean((xr - mu2) ** 2, axis=1, keepdims=True)
    res = (xr - mu2) / jnp.sqrt(var2 + 1e-5) * groot_ref[...] + beroot_ref[...]

    o_v_ref[...] = jnp.dot(res, Wv_ref[...], preferred_element_type=_f32) + bv_ref[...]
    o_cls_p_ref[...] = jnp.dot(res, Wcp_ref[...],
                               preferred_element_type=_f32) + bcp_ref[...]
    oca = jnp.dot(res, Wca_ref[...], preferred_element_type=_f32) + bca_ref[...]
    orr = jnp.dot(res, Wor_ref[...], preferred_element_type=_f32) + bor_ref[...]
    orr = jnp.where(root_m_ref[...] >= 1, -jnp.inf, orr)
    root_cat_ref[...] = jnp.concatenate([oca, orr], axis=1)

    r4_ref[...] = jnp.dot(res, Wos1_ref[...], preferred_element_type=_f32) + bos_ref[...]


def _pass2_body(partT_ref, cu_ref, r4_ref, outT_ref):
    j = pl.program_id(0)
    onehot = _onehot_rows(j, cu_ref).astype(_f32)
    gT = jax.lax.dot_general(r4_ref[...], onehot,
                             (((0,), (0,)), ((), ())),
                             preferred_element_type=_f32)
    outT_ref[...] = partT_ref[...].astype(_f32) + gT


def kernel(root_feats, root_mask, sub_feats, sub_mask, cu_seqlens,
           W_sub, b_sub, g_sub, be_sub,
           W_root, b_root, g_root, be_root,
           W_v, b_v, W_cls_p, b_cls_p, W_cls_a, b_cls_a,
           W_out_root, b_out_root, W_out_sub, b_out_sub):
    cu = cu_seqlens.astype(jnp.int32).at[0].set(0).at[-1].set(T)
    sub_maskT = jnp.transpose(sub_mask)

    row = lambda v: v.reshape(1, -1)
    first = lambda idx: (0, 0)
    tok = lambda idx: (idx, 0)
    smem = pl.BlockSpec(memory_space=pltpu.SMEM)
    full = lambda a: pl.BlockSpec(a.shape, lambda idx, n=a.ndim: (0,) * n)

    Wos2 = W_out_sub[BAG:, :]
    sums, partT = pl.pallas_call(
        _pass1_body,
        grid=(NB,),
        in_specs=[
            pl.BlockSpec((BT, FEAT_SUB), tok),
            pl.BlockSpec((MASK_SUB, BT), lambda i: (0, i)),
            smem,
            full(W_sub), pl.BlockSpec((1, BAG), first), full(Wos2),
        ],
        out_specs=(
            pl.BlockSpec((B, BAG), first),
            pl.BlockSpec((MASK_SUB, BT), lambda i: (0, i)),
        ),
        out_shape=(
            jax.ShapeDtypeStruct((B, BAG), _f32),
            jax.ShapeDtypeStruct((MASK_SUB, T), _bf),
        ),
        scratch_shapes=[pltpu.VMEM((B, BAG), _f32)],
        compiler_params=pltpu.CompilerParams(
            dimension_semantics=("arbitrary",)),
    )(sub_feats, sub_maskT, cu, W_sub, row(b_sub), Wos2)

    weights = (row(g_sub), row(be_sub), W_root, row(b_root), row(g_root),
               row(be_root), W_v, row(b_v), W_cls_p, row(b_cls_p),
               W_cls_a, row(b_cls_a), W_out_root, row(b_out_root),
               W_out_sub[0:BAG, :], row(b_out_sub))
    o_v, o_cls_p, root_cat, r4 = pl.pallas_call(
        _root_body,
        grid=(1,),
        in_specs=[full(sums), smem, full(root_feats), full(root_mask)]
                 + [full(w) for w in weights],
        out_specs=(
            pl.BlockSpec((B, 1), first),
            pl.BlockSpec((B, CLS), first),
            pl.BlockSpec((B, 1 + MASK_ROOT), first),
            pl.BlockSpec((B, MASK_SUB), first),
        ),
        out_shape=(
            jax.ShapeDtypeStruct((B, 1), _f32),
            jax.ShapeDtypeStruct((B, CLS), _f32),
            jax.ShapeDtypeStruct((B, 1 + MASK_ROOT), _f32),
            jax.ShapeDtypeStruct((B, MASK_SUB), _f32),
        ),
    )(sums, cu, root_feats, root_mask, *weights)

    out_subT = pl.pallas_call(
        _pass2_body,
        grid=(NB,),
        in_specs=[pl.BlockSpec((MASK_SUB, BT), lambda j: (0, j)), smem, full(r4)],
        out_specs=pl.BlockSpec((MASK_SUB, BT), lambda j: (0, j)),
        out_shape=jax.ShapeDtypeStruct((MASK_SUB, T), _f32),
        compiler_params=pltpu.CompilerParams(
            dimension_semantics=("parallel",)),
    )(partT, cu, r4)

    return (o_v, o_cls_p, root_cat, jnp.transpose(out_subT))


# speedup vs baseline: 1.0931x; 1.0931x over previous
import jax
import jax.numpy as jnp
from jax.experimental import pallas as pl
from jax.experimental.pallas import tpu as pltpu

B = 16
T = 32768
BAG = 128
CLS = 10
FEAT_ROOT = 192
MASK_ROOT = 2
FEAT_SUB = 128
MASK_SUB = 4

NB = 4
BT = T // NB

_bf = jnp.bfloat16
_f32 = jnp.float32


def _onehot_rows(i, cu_ref):
    t_row = jax.lax.broadcasted_iota(jnp.int32, (1, BT), 1) + i * BT
    io = jax.lax.broadcasted_iota(jnp.int32, (B, 1), 0)
    lo = jnp.zeros((B, 1), jnp.int32)
    hi = jnp.zeros((B, 1), jnp.int32)
    for s in range(B):
        lo = jnp.where(io == s, cu_ref[s], lo)
        hi = jnp.where(io == s, cu_ref[s + 1], hi)
    return (lo <= t_row) & (t_row < hi)


def _pass1_body(sub_f_ref, sub_mT_ref, cu_ref, Wsub_ref, bsub_ref, Wos2_ref,
                sums_ref, partT_ref, acc_ref):
    i = pl.program_id(0)
    x = sub_f_ref[...].astype(_bf)
    mi = jnp.transpose(sub_mT_ref[...])
    xs = (jnp.dot(x, Wsub_ref[0:FEAT_SUB, :].astype(_bf),
                  preferred_element_type=_f32)
          + jnp.dot(mi.astype(_bf), Wsub_ref[FEAT_SUB:, :].astype(_bf),
                    preferred_element_type=_f32)
          + bsub_ref[...])
    x_raw = jnp.maximum(xs, 0.01 * xs)
    x_raw_b = x_raw.astype(_bf)

    p = jnp.dot(x_raw_b, Wos2_ref[...].astype(_bf), preferred_element_type=_f32)
    partT_ref[...] = jnp.transpose(jnp.where(mi >= 1, -jnp.inf, p))

    onehot = _onehot_rows(i, cu_ref).astype(_bf)
    contrib = jnp.dot(onehot, x_raw_b, preferred_element_type=_f32)

    @pl.when(i == 0)
    def _():
        acc_ref[...] = contrib

    @pl.when(i > 0)
    def _():
        acc_ref[...] = acc_ref[...] + contrib

    @pl.when(i == NB - 1)
    def _():
        sums_ref[...] = acc_ref[...]


def _root_body(sums_ref, cu_ref, root_f_ref, root_m_ref,
               gsub_ref, besub_ref, Wroot_ref, broot_ref, groot_ref, beroot_ref,
               Wv_ref, bv_ref, Wcp_ref, bcp_ref, Wca_ref, bca_ref,
               Wor_ref, bor_ref, Wos1_ref, bos_ref,
               o_v_ref, o_cls_p_ref, root_cat_ref, r4_ref):
    io = jax.lax.broadcasted_iota(jnp.int32, (B, 1), 0)
    denom = jnp.ones((B, 1), _f32)
    nz = jnp.zeros((B, 1), _f32)
    for s in range(B):
        l = cu_ref[s + 1] - cu_ref[s]
        denom = jnp.where(io == s, jnp.maximum(l, 1).astype(_f32), denom)
        nz = jnp.where(io == s, (l > 0).astype(_f32), nz)
    x_agg = sums_ref[...] / denom * nz

    mu = jnp.mean(x_agg, axis=1, keepdims=True)
    var = jnp.mean((x_agg - mu) ** 2, axis=1, keepdims=True)
    x_agg = (x_agg - mu) / jnp.sqrt(var + 1e-5) * gsub_ref[...] + besub_ref[...]

    rm = root_m_ref[...].astype(_f32)
    xr = (jnp.dot(x_agg, Wroot_ref[0:BAG, :], preferred_element_type=_f32)
          + jnp.dot(root_f_ref[:, BAG:], Wroot_ref[BAG:FEAT_ROOT, :],
                    preferred_element_type=_f32)
          + jnp.dot(rm, Wroot_ref[FEAT_ROOT:, :], preferred_element_type=_f32)
          + broot_ref[...])
    xr = jnp.maximum(xr, 0.01 * xr)
    mu2 = jnp.mean(xr, axis=1, keepdims=True)
    var2 = jnp.mean((xr - mu2) ** 2, axis=1, keepdims=True)
    res = (xr - mu2) / jnp.sqrt(var2 + 1e-5) * groot_ref[...] + beroot_ref[...]

    o_v_ref[...] = jnp.dot(res, Wv_ref[...], preferred_element_type=_f32) + bv_ref[...]
    o_cls_p_ref[...] = jnp.dot(res, Wcp_ref[...],
                               preferred_element_type=_f32) + bcp_ref[...]
    oca = jnp.dot(res, Wca_ref[...], preferred_element_type=_f32) + bca_ref[...]
    orr = jnp.dot(res, Wor_ref[...], preferred_element_type=_f32) + bor_ref[...]
    orr = jnp.where(root_m_ref[...] >= 1, -jnp.inf, orr)
    root_cat_ref[...] = jnp.concatenate([oca, orr], axis=1)

    r4_ref[...] = jnp.dot(res, Wos1_ref[...], preferred_element_type=_f32) + bos_ref[...]


def _pass2_body(partT_ref, cu_ref, r4_ref, outT_ref):
    j = pl.program_id(0)
    onehot = _onehot_rows(j, cu_ref).astype(_f32)
    gT = jax.lax.dot_general(r4_ref[...], onehot,
                             (((0,), (0,)), ((), ())),
                             preferred_element_type=_f32)
    outT_ref[...] = partT_ref[...] + gT


def kernel(root_feats, root_mask, sub_feats, sub_mask, cu_seqlens,
           W_sub, b_sub, g_sub, be_sub,
           W_root, b_root, g_root, be_root,
           W_v, b_v, W_cls_p, b_cls_p, W_cls_a, b_cls_a,
           W_out_root, b_out_root, W_out_sub, b_out_sub):
    cu = cu_seqlens.astype(jnp.int32).at[0].set(0).at[-1].set(T)
    sub_maskT = jnp.transpose(sub_mask)

    row = lambda v: v.reshape(1, -1)
    first = lambda idx: (0, 0)
    tok = lambda idx: (idx, 0)
    smem = pl.BlockSpec(memory_space=pltpu.SMEM)
    full = lambda a: pl.BlockSpec(a.shape, lambda idx, n=a.ndim: (0,) * n)

    Wos2 = W_out_sub[BAG:, :]
    sums, partT = pl.pallas_call(
        _pass1_body,
        grid=(NB,),
        in_specs=[
            pl.BlockSpec((BT, FEAT_SUB), tok),
            pl.BlockSpec((MASK_SUB, BT), lambda i: (0, i)),
            smem,
            full(W_sub), pl.BlockSpec((1, BAG), first), full(Wos2),
        ],
        out_specs=(
            pl.BlockSpec((B, BAG), first),
            pl.BlockSpec((MASK_SUB, BT), lambda i: (0, i)),
        ),
        out_shape=(
            jax.ShapeDtypeStruct((B, BAG), _f32),
            jax.ShapeDtypeStruct((MASK_SUB, T), _f32),
        ),
        scratch_shapes=[pltpu.VMEM((B, BAG), _f32)],
        compiler_params=pltpu.CompilerParams(
            dimension_semantics=("arbitrary",)),
    )(sub_feats, sub_maskT, cu, W_sub, row(b_sub), Wos2)

    weights = (row(g_sub), row(be_sub), W_root, row(b_root), row(g_root),
               row(be_root), W_v, row(b_v), W_cls_p, row(b_cls_p),
               W_cls_a, row(b_cls_a), W_out_root, row(b_out_root),
               W_out_sub[0:BAG, :], row(b_out_sub))
    o_v, o_cls_p, root_cat, r4 = pl.pallas_call(
        _root_body,
        grid=(1,),
        in_specs=[full(sums), smem, full(root_feats), full(root_mask)]
                 + [full(w) for w in weights],
        out_specs=(
            pl.BlockSpec((B, 1), first),
            pl.BlockSpec((B, CLS), first),
            pl.BlockSpec((B, 1 + MASK_ROOT), first),
            pl.BlockSpec((B, MASK_SUB), first),
        ),
        out_shape=(
            jax.ShapeDtypeStruct((B, 1), _f32),
            jax.ShapeDtypeStruct((B, CLS), _f32),
            jax.ShapeDtypeStruct((B, 1 + MASK_ROOT), _f32),
            jax.ShapeDtypeStruct((B, MASK_SUB), _f32),
        ),
    )(sums, cu, root_feats, root_mask, *weights)

    out_subT = pl.pallas_call(
        _pass2_body,
        grid=(NB,),
        in_specs=[pl.BlockSpec((MASK_SUB, BT), lambda j: (0, j)), smem, full(r4)],
        out_specs=pl.BlockSpec((MASK_SUB, BT), lambda j: (0, j)),
        out_shape=jax.ShapeDtypeStruct((MASK_SUB, T), _f32),
        compiler_params=pltpu.CompilerParams(
            dimension_semantics=("parallel",)),
    )(partT, cu, r4)

    return (o_v, o_cls_p, root_cat, jnp.transpose(out_subT))
